# baseline (speedup 1.0000x reference)
import functools

import jax
import jax.numpy as jnp
from jax import lax
from jax.experimental import pallas as pl
from jax.experimental.pallas import tpu as pltpu
from jax.experimental.pallas import tpu_sc as plsc

NC = 2
NS = 16
NW = NC * NS
L = 16
HH = 64

_EDGE_CHUNK = 100


def _sc_degree(dst, n_pad):
    E = dst.shape[0]
    ept = E // NW
    mesh = plsc.VectorSubcoreMesh(core_axis_name="c", subcore_axis_name="s")

    @functools.partial(
        pl.kernel,
        out_type=jax.ShapeDtypeStruct((NW, n_pad), jnp.float32),
        mesh=mesh,
        scratch_types=[
            pltpu.VMEM((ept,), jnp.int32),
            pltpu.VMEM((n_pad,), jnp.float32),
        ],
        compiler_params=pltpu.CompilerParams(
            needs_layout_passes=False, use_tc_tiling_on_sc=False),
    )
    def deg_kernel(dst_hbm, out_hbm, idx_v, deg_v):
        c = lax.axis_index("c")
        s = lax.axis_index("s")
        wid = c * NS + s

        zeros16 = jnp.zeros((L,), jnp.float32)

        def zbody(i, carry):
            deg_v[pl.ds(i * L, L)] = zeros16
            return carry

        lax.fori_loop(0, n_pad // L, zbody, 0)

        pltpu.sync_copy(dst_hbm.at[pl.ds(wid * ept, ept)], idx_v)

        ones16 = jnp.ones((L,), jnp.float32)

        def body(i, carry):
            idx = idx_v[pl.ds(i * L, L)]
            plsc.addupdate_scatter(deg_v, [idx], ones16)
            return carry

        lax.fori_loop(0, ept // L, body, 0)
        pltpu.sync_copy(deg_v, out_hbm.at[wid])

    return deg_kernel(dst)


def _tc_scaled_matmul(x, W1, deg_parts):
    N, F = x.shape
    H = W1.shape[1]
    R = 512

    def body(x_ref, w_ref, deg_ref, out_ref):
        deg = jnp.sum(deg_ref[...], axis=0) + 1.0
        dis = lax.rsqrt(deg)
        h = jnp.dot(x_ref[...], w_ref[...], preferred_element_type=jnp.float32)
        g = h * dis[:, None]
        out_ref[0] = g[:, :HH]
        out_ref[1] = g[:, HH:]

    return pl.pallas_call(
        body,
        grid=(N // R,),
        in_specs=[
            pl.BlockSpec((R, F), lambda i: (i, 0)),
            pl.BlockSpec((F, H), lambda i: (0, 0)),
            pl.BlockSpec((NW, R), lambda i: (0, i)),
        ],
        out_specs=pl.BlockSpec((2, R, HH), lambda i: (0, i, 0)),
        out_shape=jax.ShapeDtypeStruct((2, N, HH), jnp.float32),
    )(x, W1, deg_parts)


def _sc_gather_scatter_add(g2, src3, dst3, n_pad):
    _, P, n_chunk, C = src3.shape
    rpt = n_pad // NS
    ZR = 16
    mesh = plsc.VectorSubcoreMesh(core_axis_name="c", subcore_axis_name="s")

    @functools.partial(
        pl.kernel,
        out_type=jax.ShapeDtypeStruct((NC, n_pad, HH), jnp.float32),
        mesh=mesh,
        scratch_types=[
            pltpu.VMEM((n_chunk, C), jnp.int32),
            pltpu.VMEM((n_chunk, C), jnp.int32),
            pltpu.VMEM((C, HH), jnp.float32),
            pltpu.VMEM((C, HH), jnp.float32),
            pltpu.VMEM((ZR, HH), jnp.float32),
            pltpu.VMEM_SHARED((n_pad, HH), jnp.float32),
            pltpu.VMEM_SHARED((n_pad, HH), jnp.float32),
            pltpu.SemaphoreType.DMA,
            pltpu.SemaphoreType.DMA,
        ],
        compiler_params=pltpu.CompilerParams(
            needs_layout_passes=False, use_tc_tiling_on_sc=False),
    )
    def gs_kernel(g2_hbm, src_hbm, dst_hbm, out_hbm,
                  src_v, dst_v, rows0, rows1, z_v, g_sh, acc_sh, sem0, sem1):
        c = lax.axis_index("c")
        s = lax.axis_index("s")

        zeros16 = jnp.zeros((L,), jnp.float32)

        def zrow(i, carry):
            def zcol(j, inner):
                z_v[i, pl.ds(j * L, L)] = zeros16
                return inner
            return lax.fori_loop(0, HH // L, zcol, carry)

        lax.fori_loop(0, ZR, zrow, 0)

        r0 = s * rpt

        def zfill(k, carry):
            pltpu.sync_copy(z_v, acc_sh.at[pl.ds(r0 + k * ZR, ZR), :])
            return carry

        lax.fori_loop(0, rpt // ZR, zfill, 0)

        pltpu.sync_copy(g2_hbm.at[c, pl.ds(r0, rpt), :],
                        g_sh.at[pl.ds(r0, rpt), :])

        plsc.subcore_barrier()

        def pass_body(p, pcarry):
            pltpu.sync_copy(src_hbm.at[s, p], src_v)
            pltpu.sync_copy(dst_hbm.at[s, p], dst_v)

            pltpu.async_copy(g_sh.at[src_v.at[0]], rows0, sem0)

            def body(b, carry):
                j0 = 2 * b
                j1 = j0 + 1
                pltpu.async_copy(g_sh.at[src_v.at[j1]], rows1, sem1)
                pltpu.make_async_copy(
                    g_sh.at[src_v.at[j0]], rows0, sem0).wait()
                pltpu.sync_copy(rows0, acc_sh.at[dst_v.at[j0]], add=True)
                j2 = jnp.minimum(j0 + 2, n_chunk - 1)
                pltpu.async_copy(g_sh.at[src_v.at[j2]], rows0, sem0)
                pltpu.make_async_copy(
                    g_sh.at[src_v.at[j1]], rows1, sem1).wait()
                pltpu.sync_copy(rows1, acc_sh.at[dst_v.at[j1]], add=True)
                return carry

            lax.fori_loop(0, n_chunk // 2, body, 0)
            pltpu.make_async_copy(g_sh.at[src_v.at[0]], rows0, sem0).wait()
            return pcarry

        lax.fori_loop(0, P, pass_body, 0)

        plsc.subcore_barrier()

        pltpu.sync_copy(acc_sh.at[pl.ds(r0, rpt), :],
                        out_hbm.at[c, pl.ds(r0, rpt), :])

    return gs_kernel(g2, src3, dst3)


def _tc_final(acc2, g2, deg_parts, b1, WfcT, bfc):
    N = g2.shape[1]
    H = WfcT.shape[0]
    R = 512

    def body(acc_ref, g_ref, deg_ref, b1_ref, w_ref, bfc_ref, out_ref):
        deg = jnp.sum(deg_ref[...], axis=0) + 1.0
        dis = lax.rsqrt(deg)
        lo = (acc_ref[0] + g_ref[0]) * dis[:, None] + b1_ref[:, :HH]
        hi = (acc_ref[1] + g_ref[1]) * dis[:, None] + b1_ref[:, HH:]
        out_ref[...] = (
            jnp.dot(lo, w_ref[:HH, :], preferred_element_type=jnp.float32)
            + jnp.dot(hi, w_ref[HH:, :], preferred_element_type=jnp.float32)
            + bfc_ref[...]
        )

    return pl.pallas_call(
        body,
        grid=(N // R,),
        in_specs=[
            pl.BlockSpec((NC, R, HH), lambda i: (0, i, 0)),
            pl.BlockSpec((NC, R, HH), lambda i: (0, i, 0)),
            pl.BlockSpec((NW, R), lambda i: (0, i)),
            pl.BlockSpec((1, H), lambda i: (0, 0)),
            pl.BlockSpec((H, H), lambda i: (0, 0)),
            pl.BlockSpec((1, H), lambda i: (0, 0)),
        ],
        out_specs=pl.BlockSpec((R, H), lambda i: (i, 0)),
        out_shape=jax.ShapeDtypeStruct((N, H), jnp.float32),
    )(acc2, g2, deg_parts, b1, WfcT, bfc)


def kernel(x, edge_index, adj, W1, b1, Wfc, bfc):
    N, F = x.shape
    E = edge_index.shape[1]
    del adj

    src = edge_index[0].astype(jnp.int32)
    dst = edge_index[1].astype(jnp.int32)

    n_pad = ((N + NS * 32 - 1) // (NS * 32)) * (NS * 32)

    ept = E // NS
    P = 2
    n_chunk = ept // (P * _EDGE_CHUNK)
    src3 = src.reshape(NS, P, n_chunk, _EDGE_CHUNK)
    dst3 = dst.reshape(NS, P, n_chunk, _EDGE_CHUNK)

    x_pad = jnp.pad(x, ((0, n_pad - N), (0, 0)))

    deg_parts = _sc_degree(dst, n_pad)
    g2 = _tc_scaled_matmul(x_pad, W1, deg_parts)
    acc2 = _sc_gather_scatter_add(g2, src3, dst3, n_pad)
    out = _tc_final(acc2, g2, deg_parts, b1.reshape(1, -1), Wfc.T,
                    bfc.reshape(1, -1))
    return out[:N]


# speedup vs baseline: 1.3117x; 1.3117x over previous
import functools

import jax
import jax.numpy as jnp
from jax import lax
from jax.experimental import pallas as pl
from jax.experimental.pallas import tpu as pltpu
from jax.experimental.pallas import tpu_sc as plsc

NC = 2
NS = 16
NW = NC * NS
L = 16

_EDGE_CHUNK = 125


def _sc_degree(dst, n_pad):
    E = dst.shape[0]
    ept = E // NW
    mesh = plsc.VectorSubcoreMesh(core_axis_name="c", subcore_axis_name="s")

    @functools.partial(
        pl.kernel,
        out_type=jax.ShapeDtypeStruct((NW, n_pad), jnp.float32),
        mesh=mesh,
        scratch_types=[
            pltpu.VMEM((ept,), jnp.int32),
            pltpu.VMEM((n_pad,), jnp.float32),
        ],
        compiler_params=pltpu.CompilerParams(
            needs_layout_passes=False, use_tc_tiling_on_sc=False),
    )
    def deg_kernel(dst_hbm, out_hbm, idx_v, deg_v):
        c = lax.axis_index("c")
        s = lax.axis_index("s")
        wid = c * NS + s

        zeros16 = jnp.zeros((L,), jnp.float32)

        def zbody(i, carry):
            deg_v[pl.ds(i * L, L)] = zeros16
            return carry

        lax.fori_loop(0, n_pad // L, zbody, 0)

        pltpu.sync_copy(dst_hbm.at[pl.ds(wid * ept, ept)], idx_v)

        ones16 = jnp.ones((L,), jnp.float32)

        def body(i, carry):
            idx = idx_v[pl.ds(i * L, L)]
            plsc.addupdate_scatter(deg_v, [idx], ones16)
            return carry

        lax.fori_loop(0, ept // L, body, 0)
        pltpu.sync_copy(deg_v, out_hbm.at[wid])

    return deg_kernel(dst)


def _tc_scaled_matmul(x, W1, deg_parts):
    N, F = x.shape
    H = W1.shape[1]
    R = 512

    def body(x_ref, w_ref, deg_ref, out_ref):
        deg = jnp.sum(deg_ref[...], axis=0) + 1.0
        dis = lax.rsqrt(deg)
        h = jnp.dot(x_ref[...], w_ref[...], preferred_element_type=jnp.float32)
        out_ref[...] = h * dis[:, None]

    return pl.pallas_call(
        body,
        grid=(N // R,),
        in_specs=[
            pl.BlockSpec((R, F), lambda i: (i, 0)),
            pl.BlockSpec((F, H), lambda i: (0, 0)),
            pl.BlockSpec((NW, R), lambda i: (0, i)),
        ],
        out_specs=pl.BlockSpec((R, H), lambda i: (i, 0)),
        out_shape=jax.ShapeDtypeStruct((N, H), jnp.float32),
    )(x, W1, deg_parts)


def _sc_gather_scatter_add(g, src4, dst4, n_pad):
    _, P, n_chunk, C = src4.shape
    H = g.shape[1]
    rpt = n_pad // NS
    ZR = 16
    mesh = plsc.VectorSubcoreMesh(core_axis_name="c", subcore_axis_name="s")

    @functools.partial(
        pl.kernel,
        out_type=jax.ShapeDtypeStruct((NC, n_pad, H), jnp.float32),
        mesh=mesh,
        scratch_types=[
            pltpu.VMEM((n_chunk, C), jnp.int32),
            pltpu.VMEM((n_chunk, C), jnp.int32),
            pltpu.VMEM((C, H), jnp.float32),
            pltpu.VMEM((C, H), jnp.float32),
            pltpu.VMEM((ZR, H), jnp.float32),
            pltpu.VMEM_SHARED((n_pad, H), jnp.float32),
            pltpu.SemaphoreType.DMA,
            pltpu.SemaphoreType.DMA,
        ],
        compiler_params=pltpu.CompilerParams(
            needs_layout_passes=False, use_tc_tiling_on_sc=False),
    )
    def gs_kernel(g_hbm, src_hbm, dst_hbm, out_hbm,
                  src_v, dst_v, rows0, rows1, z_v, acc_sh, sem0, sem1):
        c = lax.axis_index("c")
        s = lax.axis_index("s")
        wid = c * NS + s

        zeros16 = jnp.zeros((L,), jnp.float32)

        def zrow(i, carry):
            def zcol(j, inner):
                z_v[i, pl.ds(j * L, L)] = zeros16
                return inner
            return lax.fori_loop(0, H // L, zcol, carry)

        lax.fori_loop(0, ZR, zrow, 0)

        r0 = s * rpt

        def zfill(k, carry):
            pltpu.sync_copy(z_v, acc_sh.at[pl.ds(r0 + k * ZR, ZR), :])
            return carry

        lax.fori_loop(0, rpt // ZR, zfill, 0)

        plsc.subcore_barrier()

        def pass_body(p, pcarry):
            pltpu.sync_copy(src_hbm.at[wid, p], src_v)
            pltpu.sync_copy(dst_hbm.at[wid, p], dst_v)

            pltpu.async_copy(g_hbm.at[src_v.at[0]], rows0, sem0)

            def body(b, carry):
                j0 = 2 * b
                j1 = j0 + 1
                pltpu.async_copy(g_hbm.at[src_v.at[j1]], rows1, sem1)
                pltpu.make_async_copy(
                    g_hbm.at[src_v.at[j0]], rows0, sem0).wait()
                pltpu.sync_copy(rows0, acc_sh.at[dst_v.at[j0]], add=True)
                j2 = jnp.minimum(j0 + 2, n_chunk - 1)
                pltpu.async_copy(g_hbm.at[src_v.at[j2]], rows0, sem0)
                pltpu.make_async_copy(
                    g_hbm.at[src_v.at[j1]], rows1, sem1).wait()
                pltpu.sync_copy(rows1, acc_sh.at[dst_v.at[j1]], add=True)
                return carry

            lax.fori_loop(0, n_chunk // 2, body, 0)
            pltpu.make_async_copy(g_hbm.at[src_v.at[0]], rows0, sem0).wait()
            return pcarry

        lax.fori_loop(0, P, pass_body, 0)

        plsc.subcore_barrier()

        pltpu.sync_copy(acc_sh.at[pl.ds(r0, rpt), :],
                        out_hbm.at[c, pl.ds(r0, rpt), :])

    return gs_kernel(g, src4, dst4)


def _tc_final(acc, g, deg_parts, b1, WfcT, bfc):
    N, H = g.shape
    R = 512

    def body(acc_ref, g_ref, deg_ref, b1_ref, w_ref, bfc_ref, out_ref):
        deg = jnp.sum(deg_ref[...], axis=0) + 1.0
        dis = lax.rsqrt(deg)
        t = (acc_ref[0] + acc_ref[1] + g_ref[...]) * dis[:, None] + b1_ref[...]
        out_ref[...] = (
            jnp.dot(t, w_ref[...], preferred_element_type=jnp.float32)
            + bfc_ref[...]
        )

    return pl.pallas_call(
        body,
        grid=(N // R,),
        in_specs=[
            pl.BlockSpec((NC, R, H), lambda i: (0, i, 0)),
            pl.BlockSpec((R, H), lambda i: (i, 0)),
            pl.BlockSpec((NW, R), lambda i: (0, i)),
            pl.BlockSpec((1, H), lambda i: (0, 0)),
            pl.BlockSpec((H, H), lambda i: (0, 0)),
            pl.BlockSpec((1, H), lambda i: (0, 0)),
        ],
        out_specs=pl.BlockSpec((R, H), lambda i: (i, 0)),
        out_shape=jax.ShapeDtypeStruct((N, H), jnp.float32),
    )(acc, g, deg_parts, b1, WfcT, bfc)


def kernel(x, edge_index, adj, W1, b1, Wfc, bfc):
    N, F = x.shape
    E = edge_index.shape[1]
    del adj

    src = edge_index[0].astype(jnp.int32)
    dst = edge_index[1].astype(jnp.int32)

    n_pad = ((N + NS * 32 - 1) // (NS * 32)) * (NS * 32)

    ept = E // NW
    P = 2
    n_chunk = ept // (P * _EDGE_CHUNK)
    src4 = src.reshape(NW, P, n_chunk, _EDGE_CHUNK)
    dst4 = dst.reshape(NW, P, n_chunk, _EDGE_CHUNK)

    x_pad = jnp.pad(x, ((0, n_pad - N), (0, 0)))

    deg_parts = _sc_degree(dst, n_pad)
    g = _tc_scaled_matmul(x_pad, W1, deg_parts)
    acc = _sc_gather_scatter_add(g, src4, dst4, n_pad)
    out = _tc_final(acc, g, deg_parts, b1.reshape(1, -1), Wfc.T,
                    bfc.reshape(1, -1))
    return out[:N]


# speedup vs baseline: 1.3391x; 1.0209x over previous
import functools

import jax
import jax.numpy as jnp
from jax import lax
from jax.experimental import pallas as pl
from jax.experimental.pallas import tpu as pltpu
from jax.experimental.pallas import tpu_sc as plsc

NC = 2
NS = 16
NW = NC * NS
L = 16

_EDGE_CHUNK = 100


def _sc_degree(dst, n_pad):
    E = dst.shape[0]
    ept = E // NW
    mesh = plsc.VectorSubcoreMesh(core_axis_name="c", subcore_axis_name="s")

    @functools.partial(
        pl.kernel,
        out_type=jax.ShapeDtypeStruct((NW, n_pad), jnp.float32),
        mesh=mesh,
        scratch_types=[
            pltpu.VMEM((ept,), jnp.int32),
            pltpu.VMEM((n_pad,), jnp.float32),
        ],
        compiler_params=pltpu.CompilerParams(
            needs_layout_passes=False, use_tc_tiling_on_sc=False),
    )
    def deg_kernel(dst_hbm, out_hbm, idx_v, deg_v):
        c = lax.axis_index("c")
        s = lax.axis_index("s")
        wid = c * NS + s

        zeros16 = jnp.zeros((L,), jnp.float32)

        def zbody(i, carry):
            deg_v[pl.ds(i * L, L)] = zeros16
            return carry

        lax.fori_loop(0, n_pad // L, zbody, 0)

        pltpu.sync_copy(dst_hbm.at[pl.ds(wid * ept, ept)], idx_v)

        ones16 = jnp.ones((L,), jnp.float32)

        def body(i, carry):
            idx = idx_v[pl.ds(i * L, L)]
            plsc.addupdate_scatter(deg_v, [idx], ones16)
            return carry

        lax.fori_loop(0, ept // L, body, 0)
        pltpu.sync_copy(deg_v, out_hbm.at[wid])

    return deg_kernel(dst)


def _tc_scaled_matmul(x, W1, deg_parts):
    N, F = x.shape
    H = W1.shape[1]
    R = 512

    def body(x_ref, w_ref, deg_ref, out_ref):
        deg = jnp.sum(deg_ref[...], axis=0) + 1.0
        dis = lax.rsqrt(deg)
        h = jnp.dot(x_ref[...], w_ref[...], preferred_element_type=jnp.float32)
        out_ref[...] = h * dis[:, None]

    return pl.pallas_call(
        body,
        grid=(N // R,),
        in_specs=[
            pl.BlockSpec((R, F), lambda i: (i, 0)),
            pl.BlockSpec((F, H), lambda i: (0, 0)),
            pl.BlockSpec((NW, R), lambda i: (0, i)),
        ],
        out_specs=pl.BlockSpec((R, H), lambda i: (i, 0)),
        out_shape=jax.ShapeDtypeStruct((N, H), jnp.float32),
    )(x, W1, deg_parts)


def _sc_gather_scatter_add(g, src4, dst4, n_pad):
    _, P, n_chunk, C = src4.shape
    H = g.shape[1]
    rpt = n_pad // NS
    mesh = plsc.VectorSubcoreMesh(core_axis_name="c", subcore_axis_name="s")

    @functools.partial(
        pl.kernel,
        out_type=jax.ShapeDtypeStruct((NC, n_pad, H), jnp.float32),
        mesh=mesh,
        scratch_types=[
            pltpu.VMEM((n_chunk, C), jnp.int32),
            pltpu.VMEM((n_chunk, C), jnp.int32),
            pltpu.VMEM((C, H), jnp.float32),
            pltpu.VMEM((C, H), jnp.float32),
            pltpu.VMEM((C, H), jnp.float32),
            pltpu.VMEM_SHARED((n_pad, H), jnp.float32),
            pltpu.SemaphoreType.DMA,
            pltpu.SemaphoreType.DMA,
            pltpu.SemaphoreType.DMA,
            pltpu.SemaphoreType.DMA,
            pltpu.SemaphoreType.DMA,
            pltpu.SemaphoreType.DMA,
        ],
        compiler_params=pltpu.CompilerParams(
            needs_layout_passes=False, use_tc_tiling_on_sc=False),
    )
    def gs_kernel(g_hbm, src_hbm, dst_hbm, out_hbm,
                  src_v, dst_v, rows0, rows1, rows2, acc_sh,
                  sg0, sg1, sg2, ss0, ss1, ss2):
        c = lax.axis_index("c")
        s = lax.axis_index("s")
        wid = c * NS + s

        zeros16 = jnp.zeros((L,), jnp.float32)

        def zrow(i, carry):
            def zcol(j, inner):
                rows0[i, pl.ds(j * L, L)] = zeros16
                return inner
            return lax.fori_loop(0, H // L, zcol, carry)

        lax.fori_loop(0, C, zrow, 0)

        r0 = s * rpt

        def zfill(k, carry):
            pltpu.sync_copy(rows0, acc_sh.at[pl.ds(r0 + k * C, C), :])
            return carry

        nz = rpt // C
        lax.fori_loop(0, nz, zfill, 0)
        rz = rpt - nz * C
        pltpu.sync_copy(rows0.at[pl.ds(0, rz), :],
                        acc_sh.at[pl.ds(r0 + nz * C, rz), :])

        plsc.subcore_barrier()

        def gath(j, rows, sem):
            pltpu.async_copy(g_hbm.at[src_v.at[j]], rows, sem)

        def gwait(j, rows, sem):
            pltpu.make_async_copy(g_hbm.at[src_v.at[j]], rows, sem).wait()

        def scat(j, rows, sem):
            pltpu.async_copy(rows, acc_sh.at[dst_v.at[j]], sem, add=True)

        def swait(j, rows, sem):
            pltpu.make_async_copy(rows, acc_sh.at[dst_v.at[j]], sem).wait()

        def pass_body(p, pcarry):
            pltpu.sync_copy(src_hbm.at[wid, p], src_v)
            pltpu.sync_copy(dst_hbm.at[wid, p], dst_v)

            gath(0, rows0, sg0)
            gath(1, rows1, sg1)
            gath(2, rows2, sg2)
            gwait(0, rows0, sg0)
            scat(0, rows0, ss0)
            gwait(1, rows1, sg1)
            scat(1, rows1, ss1)
            swait(0, rows0, ss0)
            gath(3, rows0, sg0)

            last = n_chunk - 1

            def body(m, carry):
                k = 3 * m + 2
                gwait(k, rows2, sg2)
                scat(k, rows2, ss2)
                swait(k - 1, rows1, ss1)
                gath(jnp.minimum(k + 2, last), rows1, sg1)
                gwait(k + 1, rows0, sg0)
                scat(k + 1, rows0, ss0)
                swait(k, rows2, ss2)
                gath(jnp.minimum(k + 3, last), rows2, sg2)
                gwait(k + 2, rows1, sg1)
                scat(k + 2, rows1, ss1)
                swait(k + 1, rows0, ss0)
                gath(jnp.minimum(k + 4, last), rows0, sg0)
                return carry

            lax.fori_loop(0, (n_chunk - 2) // 3, body, 0)

            swait(last, rows1, ss1)
            gwait(last, rows2, sg2)
            gwait(last, rows0, sg0)
            return pcarry

        lax.fori_loop(0, P, pass_body, 0)

        plsc.subcore_barrier()

        pltpu.sync_copy(acc_sh.at[pl.ds(r0, rpt), :],
                        out_hbm.at[c, pl.ds(r0, rpt), :])

    return gs_kernel(g, src4, dst4)


def _tc_final(acc, g, deg_parts, b1, WfcT, bfc):
    N, H = g.shape
    R = 512

    def body(acc_ref, g_ref, deg_ref, b1_ref, w_ref, bfc_ref, out_ref):
        deg = jnp.sum(deg_ref[...], axis=0) + 1.0
        dis = lax.rsqrt(deg)
        t = (acc_ref[0] + acc_ref[1] + g_ref[...]) * dis[:, None] + b1_ref[...]
        out_ref[...] = (
            jnp.dot(t, w_ref[...], preferred_element_type=jnp.float32)
            + bfc_ref[...]
        )

    return pl.pallas_call(
        body,
        grid=(N // R,),
        in_specs=[
            pl.BlockSpec((NC, R, H), lambda i: (0, i, 0)),
            pl.BlockSpec((R, H), lambda i: (i, 0)),
            pl.BlockSpec((NW, R), lambda i: (0, i)),
            pl.BlockSpec((1, H), lambda i: (0, 0)),
            pl.BlockSpec((H, H), lambda i: (0, 0)),
            pl.BlockSpec((1, H), lambda i: (0, 0)),
        ],
        out_specs=pl.BlockSpec((R, H), lambda i: (i, 0)),
        out_shape=jax.ShapeDtypeStruct((N, H), jnp.float32),
    )(acc, g, deg_parts, b1, WfcT, bfc)


def kernel(x, edge_index, adj, W1, b1, Wfc, bfc):
    N, F = x.shape
    E = edge_index.shape[1]
    del adj

    src = edge_index[0].astype(jnp.int32)
    dst = edge_index[1].astype(jnp.int32)

    n_pad = ((N + NS * 32 - 1) // (NS * 32)) * (NS * 32)

    ept = E // NW
    P = 2
    n_chunk = ept // (P * _EDGE_CHUNK)
    src4 = src.reshape(NW, P, n_chunk, _EDGE_CHUNK)
    dst4 = dst.reshape(NW, P, n_chunk, _EDGE_CHUNK)

    x_pad = jnp.pad(x, ((0, n_pad - N), (0, 0)))

    deg_parts = _sc_degree(dst, n_pad)
    g = _tc_scaled_matmul(x_pad, W1, deg_parts)
    acc = _sc_gather_scatter_add(g, src4, dst4, n_pad)
    out = _tc_final(acc, g, deg_parts, b1.reshape(1, -1), Wfc.T,
                    bfc.reshape(1, -1))
    return out[:N]


# speedup vs baseline: 1.4868x; 1.1103x over previous
import functools

import jax
import jax.numpy as jnp
from jax import lax
from jax.experimental import pallas as pl
from jax.experimental.pallas import tpu as pltpu
from jax.experimental.pallas import tpu_sc as plsc

NC = 2
NS = 16
NW = NC * NS
L = 16

_EDGE_CHUNK = 100


def _sc_degree(dst, n_pad):
    E = dst.shape[0]
    ept = E // NW
    mesh = plsc.VectorSubcoreMesh(core_axis_name="c", subcore_axis_name="s")

    @functools.partial(
        pl.kernel,
        out_type=jax.ShapeDtypeStruct((NW, n_pad), jnp.float32),
        mesh=mesh,
        scratch_types=[
            pltpu.VMEM((ept,), jnp.int32),
            pltpu.VMEM((n_pad,), jnp.float32),
        ],
        compiler_params=pltpu.CompilerParams(
            needs_layout_passes=False, use_tc_tiling_on_sc=False),
    )
    def deg_kernel(dst_hbm, out_hbm, idx_v, deg_v):
        c = lax.axis_index("c")
        s = lax.axis_index("s")
        wid = c * NS + s

        zeros16 = jnp.zeros((L,), jnp.float32)

        def zbody(i, carry):
            deg_v[pl.ds(i * L, L)] = zeros16
            return carry

        lax.fori_loop(0, n_pad // L, zbody, 0)

        pltpu.sync_copy(dst_hbm.at[pl.ds(wid * ept, ept)], idx_v)

        ones16 = jnp.ones((L,), jnp.float32)

        def body(i, carry):
            idx = idx_v[pl.ds(i * L, L)]
            plsc.addupdate_scatter(deg_v, [idx], ones16)
            return carry

        lax.fori_loop(0, ept // L, body, 0)
        pltpu.sync_copy(deg_v, out_hbm.at[wid])

    return deg_kernel(dst)


def _tc_scaled_matmul(x, W1, deg_parts, n_pad):
    F = x.shape[1]
    H = W1.shape[1]
    R = 1024

    def body(x_ref, w_ref, deg_ref, out_ref):
        deg = jnp.sum(deg_ref[...], axis=0) + 1.0
        dis = lax.rsqrt(deg)
        h = jnp.dot(x_ref[...], w_ref[...], preferred_element_type=jnp.float32)
        out_ref[...] = h * dis[:, None]

    return pl.pallas_call(
        body,
        grid=(n_pad // R,),
        in_specs=[
            pl.BlockSpec((R, F), lambda i: (i, 0)),
            pl.BlockSpec((F, H), lambda i: (0, 0)),
            pl.BlockSpec((NW, R), lambda i: (0, i)),
        ],
        out_specs=pl.BlockSpec((R, H), lambda i: (i, 0)),
        out_shape=jax.ShapeDtypeStruct((n_pad, H), jnp.float32),
    )(x, W1, deg_parts)


def _sc_gather_scatter_add(g, src4, dst4, n_pad):
    _, P, n_chunk, C = src4.shape
    H = g.shape[1]
    rpt = n_pad // NS
    mesh = plsc.VectorSubcoreMesh(core_axis_name="c", subcore_axis_name="s")

    @functools.partial(
        pl.kernel,
        out_type=jax.ShapeDtypeStruct((NC, n_pad, H), jnp.float32),
        mesh=mesh,
        scratch_types=[
            pltpu.VMEM((n_chunk, C), jnp.int32),
            pltpu.VMEM((n_chunk, C), jnp.int32),
            pltpu.VMEM((C, H), jnp.float32),
            pltpu.VMEM((C, H), jnp.float32),
            pltpu.VMEM((C, H), jnp.float32),
            pltpu.VMEM_SHARED((n_pad, H), jnp.float32),
            pltpu.SemaphoreType.DMA,
            pltpu.SemaphoreType.DMA,
            pltpu.SemaphoreType.DMA,
            pltpu.SemaphoreType.DMA,
            pltpu.SemaphoreType.DMA,
            pltpu.SemaphoreType.DMA,
        ],
        compiler_params=pltpu.CompilerParams(
            needs_layout_passes=False, use_tc_tiling_on_sc=False),
    )
    def gs_kernel(g_hbm, src_hbm, dst_hbm, out_hbm,
                  src_v, dst_v, rows0, rows1, rows2, acc_sh,
                  sg0, sg1, sg2, ss0, ss1, ss2):
        c = lax.axis_index("c")
        s = lax.axis_index("s")
        wid = c * NS + s

        zeros16 = jnp.zeros((L,), jnp.float32)

        def zrow(i, carry):
            def zcol(j, inner):
                rows0[i, pl.ds(j * L, L)] = zeros16
                return inner
            return lax.fori_loop(0, H // L, zcol, carry)

        lax.fori_loop(0, C, zrow, 0)

        r0 = s * rpt

        def zfill(k, carry):
            pltpu.sync_copy(rows0, acc_sh.at[pl.ds(r0 + k * C, C), :])
            return carry

        nz = rpt // C
        lax.fori_loop(0, nz, zfill, 0)
        rz = rpt - nz * C
        pltpu.sync_copy(rows0.at[pl.ds(0, rz), :],
                        acc_sh.at[pl.ds(r0 + nz * C, rz), :])

        plsc.subcore_barrier()

        def gath(j, rows, sem):
            pltpu.async_copy(g_hbm.at[src_v.at[j]], rows, sem)

        def gwait(j, rows, sem):
            pltpu.make_async_copy(g_hbm.at[src_v.at[j]], rows, sem).wait()

        def scat(j, rows, sem):
            pltpu.async_copy(rows, acc_sh.at[dst_v.at[j]], sem, add=True)

        def swait(j, rows, sem):
            pltpu.make_async_copy(rows, acc_sh.at[dst_v.at[j]], sem).wait()

        def pass_body(p, pcarry):
            pltpu.sync_copy(src_hbm.at[wid, p], src_v)
            pltpu.sync_copy(dst_hbm.at[wid, p], dst_v)

            gath(0, rows0, sg0)
            gath(1, rows1, sg1)
            gath(2, rows2, sg2)
            gwait(0, rows0, sg0)
            scat(0, rows0, ss0)
            gwait(1, rows1, sg1)
            scat(1, rows1, ss1)
            swait(0, rows0, ss0)
            gath(3, rows0, sg0)

            last = n_chunk - 1

            def body(m, carry):
                k = 3 * m + 2
                gwait(k, rows2, sg2)
                scat(k, rows2, ss2)
                swait(k - 1, rows1, ss1)
                gath(jnp.minimum(k + 2, last), rows1, sg1)
                gwait(k + 1, rows0, sg0)
                scat(k + 1, rows0, ss0)
                swait(k, rows2, ss2)
                gath(jnp.minimum(k + 3, last), rows2, sg2)
                gwait(k + 2, rows1, sg1)
                scat(k + 2, rows1, ss1)
                swait(k + 1, rows0, ss0)
                gath(jnp.minimum(k + 4, last), rows0, sg0)
                return carry

            lax.fori_loop(0, (n_chunk - 2) // 3, body, 0)

            swait(last, rows1, ss1)
            gwait(last, rows2, sg2)
            gwait(last, rows0, sg0)
            return pcarry

        lax.fori_loop(0, P, pass_body, 0)

        plsc.subcore_barrier()

        pltpu.sync_copy(acc_sh.at[pl.ds(r0, rpt), :],
                        out_hbm.at[c, pl.ds(r0, rpt), :])

    return gs_kernel(g, src4, dst4)


def _tc_final(acc, g, deg_parts, b1, WfcT, bfc, n_out):
    H = g.shape[1]
    R = 1024

    def body(acc_ref, g_ref, deg_ref, b1_ref, w_ref, bfc_ref, out_ref):
        deg = jnp.sum(deg_ref[...], axis=0) + 1.0
        dis = lax.rsqrt(deg)
        t = (acc_ref[0] + acc_ref[1] + g_ref[...]) * dis[:, None] + b1_ref[...]
        out_ref[...] = (
            jnp.dot(t, w_ref[...], preferred_element_type=jnp.float32)
            + bfc_ref[...]
        )

    return pl.pallas_call(
        body,
        grid=((n_out + R - 1) // R,),
        in_specs=[
            pl.BlockSpec((NC, R, H), lambda i: (0, i, 0)),
            pl.BlockSpec((R, H), lambda i: (i, 0)),
            pl.BlockSpec((NW, R), lambda i: (0, i)),
            pl.BlockSpec((1, H), lambda i: (0, 0)),
            pl.BlockSpec((H, H), lambda i: (0, 0)),
            pl.BlockSpec((1, H), lambda i: (0, 0)),
        ],
        out_specs=pl.BlockSpec((R, H), lambda i: (i, 0)),
        out_shape=jax.ShapeDtypeStruct((n_out, H), jnp.float32),
    )(acc, g, deg_parts, b1, WfcT, bfc)


def kernel(x, edge_index, adj, W1, b1, Wfc, bfc):
    N, F = x.shape
    E = edge_index.shape[1]
    del adj

    src = edge_index[0].astype(jnp.int32)
    dst = edge_index[1].astype(jnp.int32)

    n_pad = ((N + NS * 32 - 1) // (NS * 32)) * (NS * 32)

    ept = E // NW
    P = 2
    n_chunk = ept // (P * _EDGE_CHUNK)
    src4 = src.reshape(NW, P, n_chunk, _EDGE_CHUNK)
    dst4 = dst.reshape(NW, P, n_chunk, _EDGE_CHUNK)

    deg_parts = _sc_degree(dst, n_pad)
    g = _tc_scaled_matmul(x, W1, deg_parts, n_pad)
    acc = _sc_gather_scatter_add(g, src4, dst4, n_pad)
    return _tc_final(acc, g, deg_parts, b1.reshape(1, -1), Wfc.T,
                     bfc.reshape(1, -1), N)


# speedup vs baseline: 1.5313x; 1.0299x over previous
import functools

import jax
import jax.numpy as jnp
from jax import lax
from jax.experimental import pallas as pl
from jax.experimental.pallas import tpu as pltpu
from jax.experimental.pallas import tpu_sc as plsc

NC = 2
NS = 16
NW = NC * NS
L = 16

_EDGE_CHUNK = 100


def _sc_degree(dst, n_pad):
    E = dst.shape[0]
    ept = E // NW
    mesh = plsc.VectorSubcoreMesh(core_axis_name="c", subcore_axis_name="s")

    @functools.partial(
        pl.kernel,
        out_type=jax.ShapeDtypeStruct((NW, n_pad), jnp.float32),
        mesh=mesh,
        scratch_types=[
            pltpu.VMEM((ept,), jnp.int32),
            pltpu.VMEM((n_pad,), jnp.float32),
        ],
        compiler_params=pltpu.CompilerParams(
            needs_layout_passes=False, use_tc_tiling_on_sc=False),
    )
    def deg_kernel(dst_hbm, out_hbm, idx_v, deg_v):
        c = lax.axis_index("c")
        s = lax.axis_index("s")
        wid = c * NS + s

        zeros16 = jnp.zeros((L,), jnp.float32)

        def zbody(i, carry):
            deg_v[pl.ds(i * L, L)] = zeros16
            return carry

        lax.fori_loop(0, n_pad // L, zbody, 0)

        pltpu.sync_copy(dst_hbm.at[pl.ds(wid * ept, ept)], idx_v)

        ones16 = jnp.ones((L,), jnp.float32)

        def body(i, carry):
            idx = idx_v[pl.ds(i * L, L)]
            plsc.addupdate_scatter(deg_v, [idx], ones16)
            return carry

        lax.fori_loop(0, ept // L, body, 0)
        pltpu.sync_copy(deg_v, out_hbm.at[wid])

    return deg_kernel(dst)


def _tc_scaled_matmul(x, W1, deg_parts, n_pad):
    F = x.shape[1]
    H = W1.shape[1]
    R = 2048

    def body(x_ref, w_ref, deg_ref, out_ref):
        deg = jnp.sum(deg_ref[...], axis=0) + 1.0
        dis = lax.rsqrt(deg)
        h = jnp.dot(x_ref[...], w_ref[...], preferred_element_type=jnp.float32)
        out_ref[...] = h * dis[:, None]

    return pl.pallas_call(
        body,
        grid=(n_pad // R,),
        in_specs=[
            pl.BlockSpec((R, F), lambda i: (i, 0)),
            pl.BlockSpec((F, H), lambda i: (0, 0)),
            pl.BlockSpec((NW, R), lambda i: (0, i)),
        ],
        out_specs=pl.BlockSpec((R, H), lambda i: (i, 0)),
        out_shape=jax.ShapeDtypeStruct((n_pad, H), jnp.float32),
    )(x, W1, deg_parts)


def _sc_gather_scatter_add(g, src4, dst4, n_pad):
    _, P, n_chunk, C = src4.shape
    H = g.shape[1]
    rpt = n_pad // NS
    mesh = plsc.VectorSubcoreMesh(core_axis_name="c", subcore_axis_name="s")

    @functools.partial(
        pl.kernel,
        out_type=jax.ShapeDtypeStruct((NC, n_pad, H), jnp.float32),
        mesh=mesh,
        scratch_types=[
            pltpu.VMEM((n_chunk, C), jnp.int32),
            pltpu.VMEM((n_chunk, C), jnp.int32),
            pltpu.VMEM((C, H), jnp.float32),
            pltpu.VMEM((C, H), jnp.float32),
            pltpu.VMEM((C, H), jnp.float32),
            pltpu.VMEM_SHARED((n_pad, H), jnp.float32),
            pltpu.SemaphoreType.DMA,
            pltpu.SemaphoreType.DMA,
            pltpu.SemaphoreType.DMA,
            pltpu.SemaphoreType.DMA,
            pltpu.SemaphoreType.DMA,
            pltpu.SemaphoreType.DMA,
        ],
        compiler_params=pltpu.CompilerParams(
            needs_layout_passes=False, use_tc_tiling_on_sc=False),
    )
    def gs_kernel(g_hbm, src_hbm, dst_hbm, out_hbm,
                  src_v, dst_v, rows0, rows1, rows2, acc_sh,
                  sg0, sg1, sg2, ss0, ss1, ss2):
        c = lax.axis_index("c")
        s = lax.axis_index("s")
        wid = c * NS + s

        zeros16 = jnp.zeros((L,), jnp.float32)

        def zrow(i, carry):
            def zcol(j, inner):
                rows0[i, pl.ds(j * L, L)] = zeros16
                return inner
            return lax.fori_loop(0, H // L, zcol, carry)

        lax.fori_loop(0, C, zrow, 0)

        r0 = s * rpt

        def zfill(k, carry):
            pltpu.sync_copy(rows0, acc_sh.at[pl.ds(r0 + k * C, C), :])
            return carry

        nz = rpt // C
        lax.fori_loop(0, nz, zfill, 0)
        rz = rpt - nz * C
        pltpu.sync_copy(rows0.at[pl.ds(0, rz), :],
                        acc_sh.at[pl.ds(r0 + nz * C, rz), :])

        plsc.subcore_barrier()

        def gath(j, rows, sem):
            pltpu.async_copy(g_hbm.at[src_v.at[j]], rows, sem)

        def gwait(j, rows, sem):
            pltpu.make_async_copy(g_hbm.at[src_v.at[j]], rows, sem).wait()

        def scat(j, rows, sem):
            pltpu.async_copy(rows, acc_sh.at[dst_v.at[j]], sem, add=True)

        def swait(j, rows, sem):
            pltpu.make_async_copy(rows, acc_sh.at[dst_v.at[j]], sem).wait()

        def pass_body(p, pcarry):
            pltpu.sync_copy(src_hbm.at[wid, p], src_v)
            pltpu.sync_copy(dst_hbm.at[wid, p], dst_v)

            gath(0, rows0, sg0)
            gath(1, rows1, sg1)
            gath(2, rows2, sg2)
            gwait(0, rows0, sg0)
            scat(0, rows0, ss0)
            gwait(1, rows1, sg1)
            scat(1, rows1, ss1)
            swait(0, rows0, ss0)
            gath(3, rows0, sg0)

            last = n_chunk - 1

            def body(m, carry):
                k = 3 * m + 2
                gwait(k, rows2, sg2)
                scat(k, rows2, ss2)
                swait(k - 1, rows1, ss1)
                gath(jnp.minimum(k + 2, last), rows1, sg1)
                gwait(k + 1, rows0, sg0)
                scat(k + 1, rows0, ss0)
                swait(k, rows2, ss2)
                gath(jnp.minimum(k + 3, last), rows2, sg2)
                gwait(k + 2, rows1, sg1)
                scat(k + 2, rows1, ss1)
                swait(k + 1, rows0, ss0)
                gath(jnp.minimum(k + 4, last), rows0, sg0)
                return carry

            lax.fori_loop(0, (n_chunk - 2) // 3, body, 0)

            swait(last, rows1, ss1)
            gwait(last, rows2, sg2)
            gwait(last, rows0, sg0)
            return pcarry

        lax.fori_loop(0, P, pass_body, 0)

        plsc.subcore_barrier()

        pltpu.sync_copy(acc_sh.at[pl.ds(r0, rpt), :],
                        out_hbm.at[c, pl.ds(r0, rpt), :])

    return gs_kernel(g, src4, dst4)


def _tc_final(acc, g, deg_parts, b1, WfcT, bfc, n_out):
    H = g.shape[1]
    R = 2048

    def body(acc_ref, g_ref, deg_ref, b1_ref, w_ref, bfc_ref, out_ref):
        deg = jnp.sum(deg_ref[...], axis=0) + 1.0
        dis = lax.rsqrt(deg)
        t = (acc_ref[0] + acc_ref[1] + g_ref[...]) * dis[:, None] + b1_ref[...]
        out_ref[...] = (
            jnp.dot(t, w_ref[...], preferred_element_type=jnp.float32)
            + bfc_ref[...]
        )

    return pl.pallas_call(
        body,
        grid=((n_out + R - 1) // R,),
        in_specs=[
            pl.BlockSpec((NC, R, H), lambda i: (0, i, 0)),
            pl.BlockSpec((R, H), lambda i: (i, 0)),
            pl.BlockSpec((NW, R), lambda i: (0, i)),
            pl.BlockSpec((1, H), lambda i: (0, 0)),
            pl.BlockSpec((H, H), lambda i: (0, 0)),
            pl.BlockSpec((1, H), lambda i: (0, 0)),
        ],
        out_specs=pl.BlockSpec((R, H), lambda i: (i, 0)),
        out_shape=jax.ShapeDtypeStruct((n_out, H), jnp.float32),
    )(acc, g, deg_parts, b1, WfcT, bfc)


def kernel(x, edge_index, adj, W1, b1, Wfc, bfc):
    N, F = x.shape
    E = edge_index.shape[1]
    del adj

    src = edge_index[0].astype(jnp.int32)
    dst = edge_index[1].astype(jnp.int32)

    n_pad = ((N + NS * 32 - 1) // (NS * 32)) * (NS * 32)

    ept = E // NW
    P = 2
    n_chunk = ept // (P * _EDGE_CHUNK)
    src4 = src.reshape(NW, P, n_chunk, _EDGE_CHUNK)
    dst4 = dst.reshape(NW, P, n_chunk, _EDGE_CHUNK)

    deg_parts = _sc_degree(dst, n_pad)
    g = _tc_scaled_matmul(x, W1, deg_parts, n_pad)
    acc = _sc_gather_scatter_add(g, src4, dst4, n_pad)
    return _tc_final(acc, g, deg_parts, b1.reshape(1, -1), Wfc.T,
                     bfc.reshape(1, -1), N)


# speedup vs baseline: 1.5594x; 1.0183x over previous
import functools

import jax
import jax.numpy as jnp
from jax import lax
from jax.experimental import pallas as pl
from jax.experimental.pallas import tpu as pltpu
from jax.experimental.pallas import tpu_sc as plsc

NC = 2
NS = 16
NW = NC * NS
L = 16

_EDGE_CHUNK = 100


def _sc_degree(dst, n_pad):
    E = dst.shape[0]
    ept = E // NW
    mesh = plsc.VectorSubcoreMesh(core_axis_name="c", subcore_axis_name="s")

    @functools.partial(
        pl.kernel,
        out_type=jax.ShapeDtypeStruct((NW, n_pad), jnp.float32),
        mesh=mesh,
        scratch_types=[
            pltpu.VMEM((ept,), jnp.int32),
            pltpu.VMEM((n_pad,), jnp.float32),
        ],
        compiler_params=pltpu.CompilerParams(
            needs_layout_passes=False, use_tc_tiling_on_sc=False),
    )
    def deg_kernel(dst_hbm, out_hbm, idx_v, deg_v):
        c = lax.axis_index("c")
        s = lax.axis_index("s")
        wid = c * NS + s

        zeros16 = jnp.zeros((L,), jnp.float32)

        def zbody(i, carry):
            deg_v[pl.ds(i * L, L)] = zeros16
            return carry

        lax.fori_loop(0, n_pad // L, zbody, 0)

        pltpu.sync_copy(dst_hbm.at[pl.ds(wid * ept, ept)], idx_v)

        ones16 = jnp.ones((L,), jnp.float32)

        def body(i, carry):
            idx = idx_v[pl.ds(i * L, L)]
            plsc.addupdate_scatter(deg_v, [idx], ones16)
            return carry

        lax.fori_loop(0, ept // L, body, 0)
        pltpu.sync_copy(deg_v, out_hbm.at[wid])

    return deg_kernel(dst)


def _tc_scaled_matmul(x, W1, deg_parts, n_pad):
    F = x.shape[1]
    H = W1.shape[1]
    R = 5120

    def body(x_ref, w_ref, deg_ref, out_ref):
        deg = jnp.sum(deg_ref[...], axis=0) + 1.0
        dis = lax.rsqrt(deg)
        h = jnp.dot(x_ref[...], w_ref[...], preferred_element_type=jnp.float32)
        out_ref[...] = h * dis[:, None]

    return pl.pallas_call(
        body,
        grid=(n_pad // R,),
        in_specs=[
            pl.BlockSpec((R, F), lambda i: (i, 0)),
            pl.BlockSpec((F, H), lambda i: (0, 0)),
            pl.BlockSpec((NW, R), lambda i: (0, i)),
        ],
        out_specs=pl.BlockSpec((R, H), lambda i: (i, 0)),
        out_shape=jax.ShapeDtypeStruct((n_pad, H), jnp.float32),
    )(x, W1, deg_parts)


def _sc_gather_scatter_add(g, src4, dst4, n_pad):
    _, P, n_chunk, C = src4.shape
    H = g.shape[1]
    rpt = n_pad // NS
    mesh = plsc.VectorSubcoreMesh(core_axis_name="c", subcore_axis_name="s")

    @functools.partial(
        pl.kernel,
        out_type=jax.ShapeDtypeStruct((NC, n_pad, H), jnp.float32),
        mesh=mesh,
        scratch_types=[
            pltpu.VMEM((n_chunk, C), jnp.int32),
            pltpu.VMEM((n_chunk, C), jnp.int32),
            pltpu.VMEM((C, H), jnp.float32),
            pltpu.VMEM((C, H), jnp.float32),
            pltpu.VMEM((C, H), jnp.float32),
            pltpu.VMEM_SHARED((n_pad, H), jnp.float32),
            pltpu.SemaphoreType.DMA,
            pltpu.SemaphoreType.DMA,
            pltpu.SemaphoreType.DMA,
            pltpu.SemaphoreType.DMA,
            pltpu.SemaphoreType.DMA,
            pltpu.SemaphoreType.DMA,
        ],
        compiler_params=pltpu.CompilerParams(
            needs_layout_passes=False, use_tc_tiling_on_sc=False),
    )
    def gs_kernel(g_hbm, src_hbm, dst_hbm, out_hbm,
                  src_v, dst_v, rows0, rows1, rows2, acc_sh,
                  sg0, sg1, sg2, ss0, ss1, ss2):
        c = lax.axis_index("c")
        s = lax.axis_index("s")
        wid = c * NS + s

        zeros16 = jnp.zeros((L,), jnp.float32)

        def zrow(i, carry):
            def zcol(j, inner):
                rows0[i, pl.ds(j * L, L)] = zeros16
                return inner
            return lax.fori_loop(0, H // L, zcol, carry)

        lax.fori_loop(0, C, zrow, 0)

        r0 = s * rpt

        def zfill(k, carry):
            pltpu.sync_copy(rows0, acc_sh.at[pl.ds(r0 + k * C, C), :])
            return carry

        nz = rpt // C
        lax.fori_loop(0, nz, zfill, 0)
        rz = rpt - nz * C
        pltpu.sync_copy(rows0.at[pl.ds(0, rz), :],
                        acc_sh.at[pl.ds(r0 + nz * C, rz), :])

        plsc.subcore_barrier()

        def gath(j, rows, sem):
            pltpu.async_copy(g_hbm.at[src_v.at[j]], rows, sem)

        def gwait(j, rows, sem):
            pltpu.make_async_copy(g_hbm.at[src_v.at[j]], rows, sem).wait()

        def scat(j, rows, sem):
            pltpu.async_copy(rows, acc_sh.at[dst_v.at[j]], sem, add=True)

        def swait(j, rows, sem):
            pltpu.make_async_copy(rows, acc_sh.at[dst_v.at[j]], sem).wait()

        def pass_body(p, pcarry):
            pltpu.sync_copy(src_hbm.at[wid, p], src_v)
            pltpu.sync_copy(dst_hbm.at[wid, p], dst_v)

            gath(0, rows0, sg0)
            gath(1, rows1, sg1)
            gath(2, rows2, sg2)
            gwait(0, rows0, sg0)
            scat(0, rows0, ss0)
            gwait(1, rows1, sg1)
            scat(1, rows1, ss1)
            swait(0, rows0, ss0)
            gath(3, rows0, sg0)

            last = n_chunk - 1

            def body(m, carry):
                k = 3 * m + 2
                gwait(k, rows2, sg2)
                scat(k, rows2, ss2)
                swait(k - 1, rows1, ss1)
                gath(jnp.minimum(k + 2, last), rows1, sg1)
                gwait(k + 1, rows0, sg0)
                scat(k + 1, rows0, ss0)
                swait(k, rows2, ss2)
                gath(jnp.minimum(k + 3, last), rows2, sg2)
                gwait(k + 2, rows1, sg1)
                scat(k + 2, rows1, ss1)
                swait(k + 1, rows0, ss0)
                gath(jnp.minimum(k + 4, last), rows0, sg0)
                return carry

            lax.fori_loop(0, (n_chunk - 2) // 3, body, 0)

            swait(last, rows1, ss1)
            gwait(last, rows2, sg2)
            gwait(last, rows0, sg0)
            return pcarry

        lax.fori_loop(0, P, pass_body, 0)

        plsc.subcore_barrier()

        pltpu.sync_copy(acc_sh.at[pl.ds(r0, rpt), :],
                        out_hbm.at[c, pl.ds(r0, rpt), :])

    return gs_kernel(g, src4, dst4)


def _tc_final(acc, g, deg_parts, b1, WfcT, bfc, n_out):
    H = g.shape[1]
    R = 5120

    def body(acc_ref, g_ref, deg_ref, b1_ref, w_ref, bfc_ref, out_ref):
        deg = jnp.sum(deg_ref[...], axis=0) + 1.0
        dis = lax.rsqrt(deg)
        t = (acc_ref[0] + acc_ref[1] + g_ref[...]) * dis[:, None] + b1_ref[...]
        out_ref[...] = (
            jnp.dot(t, w_ref[...], preferred_element_type=jnp.float32)
            + bfc_ref[...]
        )

    return pl.pallas_call(
        body,
        grid=((n_out + R - 1) // R,),
        in_specs=[
            pl.BlockSpec((NC, R, H), lambda i: (0, i, 0)),
            pl.BlockSpec((R, H), lambda i: (i, 0)),
            pl.BlockSpec((NW, R), lambda i: (0, i)),
            pl.BlockSpec((1, H), lambda i: (0, 0)),
            pl.BlockSpec((H, H), lambda i: (0, 0)),
            pl.BlockSpec((1, H), lambda i: (0, 0)),
        ],
        out_specs=pl.BlockSpec((R, H), lambda i: (i, 0)),
        out_shape=jax.ShapeDtypeStruct((n_out, H), jnp.float32),
    )(acc, g, deg_parts, b1, WfcT, bfc)


def kernel(x, edge_index, adj, W1, b1, Wfc, bfc):
    N, F = x.shape
    E = edge_index.shape[1]
    del adj

    src = edge_index[0].astype(jnp.int32)
    dst = edge_index[1].astype(jnp.int32)

    n_pad = ((N + NS * 32 - 1) // (NS * 32)) * (NS * 32)

    ept = E // NW
    P = 2
    n_chunk = ept // (P * _EDGE_CHUNK)
    src4 = src.reshape(NW, P, n_chunk, _EDGE_CHUNK)
    dst4 = dst.reshape(NW, P, n_chunk, _EDGE_CHUNK)

    deg_parts = _sc_degree(dst, n_pad)
    g = _tc_scaled_matmul(x, W1, deg_parts, n_pad)
    acc = _sc_gather_scatter_add(g, src4, dst4, n_pad)
    return _tc_final(acc, g, deg_parts, b1.reshape(1, -1), Wfc.T,
                     bfc.reshape(1, -1), N)
